# baseline (device time: 11739 ns/iter reference)
import jax
import jax.numpy as jnp
from jax import lax
from jax.experimental import pallas as pl
from jax.experimental.pallas import tpu as pltpu

N_DEV = 4
EPS = 1e-5
COMM = True


def kernel(x, t_emb, W_scale, W_shift):
    b, s, c_local = x.shape
    c_global = c_local * N_DEV

    def body(x_ref, t_ref, ws_ref, wsh_ref, out_ref,
             comm_ref, send_sems, recv_sems):
        my_pos = lax.axis_index("i")

        if COMM:
            barrier_sem = pltpu.get_barrier_semaphore()
            for k in range(1, N_DEV):
                pl.semaphore_signal(
                    barrier_sem, inc=1,
                    device_id=((my_pos + k) % N_DEV,),
                    device_id_type=pl.DeviceIdType.MESH,
                )

        xv = x_ref[:, :, :]
        s1 = jnp.sum(xv, axis=-1)
        s2 = jnp.sum(xv * xv, axis=-1)
        comm_ref[0] = jnp.stack([s1, s2])

        rdmas = []
        if COMM:
            pl.semaphore_wait(barrier_sem, N_DEV - 1)

            for k in range(1, N_DEV):
                rdma = pltpu.make_async_remote_copy(
                    src_ref=comm_ref.at[0],
                    dst_ref=comm_ref.at[k],
                    send_sem=send_sems.at[k - 1],
                    recv_sem=recv_sems.at[k - 1],
                    device_id=((my_pos + k) % N_DEV,),
                    device_id_type=pl.DeviceIdType.MESH,
                )
                rdma.start()
                rdmas.append(rdma)

        tv = t_ref[:, :]
        scale = jnp.dot(tv, ws_ref[:, :], preferred_element_type=jnp.float32)
        shift = jnp.dot(tv, wsh_ref[:, :], preferred_element_type=jnp.float32)

        for rdma in rdmas:
            rdma.wait_recv()
        for rdma in rdmas:
            rdma.wait_send()

        if COMM:
            stats = (comm_ref[0] + comm_ref[1]) + (comm_ref[2] + comm_ref[3])
        else:
            stats = comm_ref[0] * 4.0
        mean = stats[0] / c_global
        var = stats[1] / c_global - mean * mean
        inv = lax.rsqrt(var + EPS)

        h = (xv - mean[:, :, None]) * inv[:, :, None]
        out = h * (1.0 + scale[:, None, :]) + shift[:, None, :]
        out_ref[:, :, :] = out.astype(jnp.bfloat16)

    return pl.pallas_call(
        body,
        out_shape=jax.ShapeDtypeStruct((b, s, c_local), jnp.bfloat16),
        in_specs=[pl.BlockSpec(memory_space=pltpu.VMEM)] * 4,
        out_specs=pl.BlockSpec(memory_space=pltpu.VMEM),
        scratch_shapes=[
            pltpu.VMEM((N_DEV, 2, b, s), jnp.float32),
            pltpu.SemaphoreType.DMA((N_DEV - 1,)),
            pltpu.SemaphoreType.DMA((N_DEV - 1,)),
        ],
        compiler_params=(
            pltpu.CompilerParams(collective_id=0) if COMM
            else pltpu.CompilerParams()
        ),
    )(x, t_emb, W_scale, W_shift)


# device time: 11480 ns/iter; 1.0226x vs baseline; 1.0226x over previous
import jax
import jax.numpy as jnp
from jax import lax
from jax.experimental import pallas as pl
from jax.experimental.pallas import tpu as pltpu

N_DEV = 4
EPS = 1e-5
N_CHUNK = 2


def kernel(x, t_emb, W_scale, W_shift):
    b, s, c_local = x.shape
    c_global = c_local * N_DEV
    cs = s // N_CHUNK

    def body(x_ref, t_ref, ws_ref, wsh_ref, out_ref,
             comm_ref, send_sems, recv_sems):
        my_pos = lax.axis_index("i")

        barrier_sem = pltpu.get_barrier_semaphore()
        for k in range(1, N_DEV):
            pl.semaphore_signal(
                barrier_sem, inc=1,
                device_id=((my_pos + k) % N_DEV,),
                device_id_type=pl.DeviceIdType.MESH,
            )

        rdmas = []

        def send_chunk(j):
            xj = x_ref[:, pl.ds(j * cs, cs), :]
            s1 = jnp.sum(xj, axis=-1)
            s2 = jnp.sum(xj * xj, axis=-1)
            comm_ref[j, 0] = jnp.stack([s1, s2])
            sends = []
            for k in range(1, N_DEV):
                rdma = pltpu.make_async_remote_copy(
                    src_ref=comm_ref.at[j, 0],
                    dst_ref=comm_ref.at[j, k],
                    send_sem=send_sems.at[j, k - 1],
                    recv_sem=recv_sems.at[j, k - 1],
                    device_id=((my_pos + k) % N_DEV,),
                    device_id_type=pl.DeviceIdType.MESH,
                )
                rdma.start()
                sends.append(rdma)
            return sends

        chunk_rdmas = []
        for j in range(N_CHUNK):
            if j == 0:
                xj = x_ref[:, pl.ds(0, cs), :]
                s1 = jnp.sum(xj, axis=-1)
                s2 = jnp.sum(xj * xj, axis=-1)
                comm_ref[0, 0] = jnp.stack([s1, s2])
                pl.semaphore_wait(barrier_sem, N_DEV - 1)
                sends = []
                for k in range(1, N_DEV):
                    rdma = pltpu.make_async_remote_copy(
                        src_ref=comm_ref.at[0, 0],
                        dst_ref=comm_ref.at[0, k],
                        send_sem=send_sems.at[0, k - 1],
                        recv_sem=recv_sems.at[0, k - 1],
                        device_id=((my_pos + k) % N_DEV,),
                        device_id_type=pl.DeviceIdType.MESH,
                    )
                    rdma.start()
                    sends.append(rdma)
                chunk_rdmas.append(sends)
            else:
                chunk_rdmas.append(send_chunk(j))

        tv = t_ref[:, :]
        scale = jnp.dot(tv, ws_ref[:, :], preferred_element_type=jnp.float32)
        shift = jnp.dot(tv, wsh_ref[:, :], preferred_element_type=jnp.float32)
        scale1 = (1.0 + scale)[:, None, :]
        shift1 = shift[:, None, :]

        for j in range(N_CHUNK):
            for rdma in chunk_rdmas[j]:
                rdma.wait_recv()
            stats = (comm_ref[j, 0] + comm_ref[j, 1]) + (
                comm_ref[j, 2] + comm_ref[j, 3])
            mean = stats[0] / c_global
            var = stats[1] / c_global - mean * mean
            inv = lax.rsqrt(var + EPS)
            xj = x_ref[:, pl.ds(j * cs, cs), :]
            h = (xj - mean[:, :, None]) * inv[:, :, None]
            out = h * scale1 + shift1
            out_ref[:, pl.ds(j * cs, cs), :] = out.astype(jnp.bfloat16)

        for sends in chunk_rdmas:
            for rdma in sends:
                rdma.wait_send()

    return pl.pallas_call(
        body,
        out_shape=jax.ShapeDtypeStruct((b, s, c_local), jnp.bfloat16),
        in_specs=[pl.BlockSpec(memory_space=pltpu.VMEM)] * 4,
        out_specs=pl.BlockSpec(memory_space=pltpu.VMEM),
        scratch_shapes=[
            pltpu.VMEM((N_CHUNK, N_DEV, 2, b, cs), jnp.float32),
            pltpu.SemaphoreType.DMA((N_CHUNK, N_DEV - 1)),
            pltpu.SemaphoreType.DMA((N_CHUNK, N_DEV - 1)),
        ],
        compiler_params=pltpu.CompilerParams(collective_id=0),
    )(x, t_emb, W_scale, W_shift)
